# baseline (speedup 1.0000x reference)
import dataclasses
import functools

import jax
import jax.numpy as jnp
from jax import lax
from jax.experimental import pallas as pl
from jax.experimental.pallas import tpu as pltpu
from jax.experimental.pallas import tpu_sc as plsc

EPS = 1e-8
LANES = 16
NUM_CORES = 2
NUM_SUBCORES = 16
NW = NUM_CORES * NUM_SUBCORES
CHUNK = 64
NBUF = 3


def _normalize_body(x_ref, o_ref):
    x = x_ref[...]
    ss = jnp.sum(x * x, axis=1, keepdims=True)
    inv = 1.0 / jnp.maximum(jnp.sqrt(ss), EPS)
    o_ref[...] = x * inv


def _finalize_body(s_ref, d_ref, o_ref):
    s = jnp.sum(s_ref[...], axis=0, keepdims=True)
    d = jnp.sum(d_ref[...], axis=0, keepdims=True)
    o_ref[...] = jnp.where(d > 0.0, s / jnp.maximum(d, 1.0), 1.0)


@functools.lru_cache(maxsize=None)
def _make_edge_kernel(n_nodes, d, ch, n_edges):
    nseg = d // LANES
    n_pad = -(-n_nodes // 2048) * 2048
    mesh = plsc.VectorSubcoreMesh(core_axis_name="c", subcore_axis_name="s")
    out_t = (
        jax.ShapeDtypeStruct((NUM_CORES, n_pad), jnp.float32),
        jax.ShapeDtypeStruct((NUM_CORES, n_pad), jnp.float32),
    )

    cp = pltpu.CompilerParams()
    if "needs_layout_passes" in pltpu.CompilerParams.__dataclass_fields__:
        cp = dataclasses.replace(cp, needs_layout_passes=False)

    @functools.partial(
        pl.kernel,
        out_type=out_t,
        mesh=mesh,
        compiler_params=cp,
        scratch_types=[
            pltpu.VMEM((ch, CHUNK), jnp.int32),
            pltpu.VMEM((ch, CHUNK), jnp.int32),
            pltpu.VMEM((CHUNK,), jnp.float32),
            pltpu.VMEM((NBUF, CHUNK, d), jnp.float32),
            pltpu.VMEM((NBUF, CHUNK, d), jnp.float32),
            pltpu.VMEM((CHUNK,), jnp.float32),
            pltpu.VMEM((LANES * LANES,), jnp.float32),
            pltpu.VMEM((2048,), jnp.float32),
            pltpu.VMEM_SHARED((n_pad,), jnp.float32),
            pltpu.VMEM_SHARED((n_pad,), jnp.float32),
            pltpu.SemaphoreType.DMA,
            pltpu.SemaphoreType.DMA,
        ],
    )
    def edge_kernel(xhat_hbm, src_hbm, dst_hbm, sums_hbm, degs_hbm,
                    src_v, dst_v, val_c, srows, drows, sim_v, tmp_v, stage_v,
                    shared_sum, shared_deg, sem_a, sem_b):
        cid = lax.axis_index("c")
        sid = lax.axis_index("s")
        wid = sid * NUM_CORES + cid
        zeros16 = jnp.zeros((LANES,), jnp.float32)
        lane_iota = lax.iota(jnp.int32, LANES)

        @pl.when(sid == 0)
        def _init():
            @pl.loop(0, 2048, step=LANES)
            def _z(i):
                stage_v[pl.ds(pl.multiple_of(i, LANES), LANES)] = zeros16

            @pl.loop(0, n_pad, step=2048)
            def _zs(i):
                ii = pl.multiple_of(i, 2048)
                pltpu.sync_copy(stage_v, shared_sum.at[pl.ds(ii, 2048)])
                pltpu.sync_copy(stage_v, shared_deg.at[pl.ds(ii, 2048)])

        pltpu.sync_copy(src_hbm.at[wid], src_v)
        pltpu.sync_copy(dst_hbm.at[wid], dst_v)

        for b in range(NBUF):
            pltpu.async_copy(xhat_hbm.at[src_v.at[b]], srows.at[b], sem_a)
            pltpu.async_copy(xhat_hbm.at[dst_v.at[b]], drows.at[b], sem_b)
        plsc.subcore_barrier()

        @pl.loop(0, ch, step=NBUF)
        def _ring(g):
            for b in range(NBUF):
                j = g + b
                srow = srows.at[b]
                drow = drows.at[b]
                pltpu.make_async_copy(xhat_hbm.at[src_v.at[j]], srow, sem_a).wait()
                pltpu.make_async_copy(xhat_hbm.at[dst_v.at[j]], drow, sem_b).wait()

                chunk_eid = (wid * ch + j) * CHUNK

                @pl.loop(0, CHUNK // LANES)
                def _group(g2):
                    base = pl.multiple_of(g2 * LANES, LANES)
                    vmask = jnp.where(chunk_eid + base + lane_iota < n_edges,
                                      1.0, 0.0).astype(jnp.float32)
                    for rr in range(LANES):
                        a = srow[base + rr, pl.ds(0, LANES)]
                        bb = drow[base + rr, pl.ds(0, LANES)]
                        acc = a * bb
                        for kk in range(1, nseg):
                            a = srow[base + rr, pl.ds(kk * LANES, LANES)]
                            bb = drow[base + rr, pl.ds(kk * LANES, LANES)]
                            acc = acc + a * bb
                        tmp_v[pl.ds(rr * LANES, LANES)] = acc
                    col = zeros16
                    for c in range(LANES):
                        col = col + plsc.load_gather(
                            tmp_v, [lane_iota * LANES + c])
                    sim_v[pl.ds(base, LANES)] = col * vmask
                    val_c[pl.ds(base, LANES)] = vmask

                pltpu.sync_copy(sim_v, shared_sum.at[src_v.at[j]], add=True)
                pltpu.sync_copy(sim_v, shared_sum.at[dst_v.at[j]], add=True)
                pltpu.sync_copy(val_c, shared_deg.at[src_v.at[j]], add=True)
                pltpu.sync_copy(val_c, shared_deg.at[dst_v.at[j]], add=True)

                @pl.when(j + NBUF < ch)
                def _prefetch():
                    pltpu.async_copy(xhat_hbm.at[src_v.at[j + NBUF]], srows.at[b], sem_a)
                    pltpu.async_copy(xhat_hbm.at[dst_v.at[j + NBUF]], drows.at[b], sem_b)

        plsc.subcore_barrier()

        @pl.when(sid == 0)
        def _drain():
            @pl.loop(0, n_pad, step=2048)
            def _d(i):
                ii = pl.multiple_of(i, 2048)
                pltpu.sync_copy(shared_sum.at[pl.ds(ii, 2048)], stage_v)
                pltpu.sync_copy(stage_v, sums_hbm.at[cid].at[pl.ds(ii, 2048)])
                pltpu.sync_copy(shared_deg.at[pl.ds(ii, 2048)], stage_v)
                pltpu.sync_copy(stage_v, degs_hbm.at[cid].at[pl.ds(ii, 2048)])

    return edge_kernel


def kernel(node_features, edge_index):
    n, d = node_features.shape
    e = edge_index.shape[1]

    xhat = pl.pallas_call(
        _normalize_body,
        out_shape=jax.ShapeDtypeStruct((n, d), jnp.float32),
    )(node_features)

    ch = -(-e // (NW * CHUNK))
    ch = NBUF * (-(-ch // NBUF))
    ep = NW * CHUNK * ch
    pad = ep - e
    src = edge_index[0].astype(jnp.int32)
    dst = edge_index[1].astype(jnp.int32)
    srcp = jnp.pad(src, (0, pad)).reshape(NW, ch, CHUNK)
    dstp = jnp.pad(dst, (0, pad)).reshape(NW, ch, CHUNK)

    sums, degs = _make_edge_kernel(n, d, ch, e)(xhat, srcp, dstp)

    n_pad = sums.shape[1]
    out = pl.pallas_call(
        _finalize_body,
        out_shape=jax.ShapeDtypeStruct((1, n_pad), jnp.float32),
    )(sums, degs)
    return out.reshape(n_pad)[:n]


# speedup vs baseline: 1.0002x; 1.0002x over previous
import dataclasses
import functools

import jax
import jax.numpy as jnp
from jax import lax
from jax.experimental import pallas as pl
from jax.experimental.pallas import tpu as pltpu
from jax.experimental.pallas import tpu_sc as plsc

EPS = 1e-8
LANES = 16
NUM_CORES = 2
NUM_SUBCORES = 16
NW = NUM_CORES * NUM_SUBCORES
CHUNK = 64
NBUF = 3


def _normalize_body(x_ref, o_ref):
    x = x_ref[...]
    ss = jnp.sum(x * x, axis=1, keepdims=True)
    inv = 1.0 / jnp.maximum(jnp.sqrt(ss), EPS)
    o_ref[...] = x * inv


def _finalize_body(s_ref, d_ref, o_ref):
    s = jnp.sum(s_ref[...], axis=0, keepdims=True)
    d = jnp.sum(d_ref[...], axis=0, keepdims=True)
    o_ref[...] = jnp.where(d > 0.0, s / jnp.maximum(d, 1.0), 1.0)


@functools.lru_cache(maxsize=None)
def _make_edge_kernel(n_nodes, d, ch, n_edges):
    nseg = d // LANES
    n_pad = -(-n_nodes // 2048) * 2048
    mesh = plsc.VectorSubcoreMesh(core_axis_name="c", subcore_axis_name="s")
    out_t = (
        jax.ShapeDtypeStruct((NUM_CORES, n_pad), jnp.float32),
        jax.ShapeDtypeStruct((NUM_CORES, n_pad), jnp.float32),
    )

    cp = pltpu.CompilerParams()
    if "needs_layout_passes" in pltpu.CompilerParams.__dataclass_fields__:
        cp = dataclasses.replace(cp, needs_layout_passes=False)

    @functools.partial(
        pl.kernel,
        out_type=out_t,
        mesh=mesh,
        compiler_params=cp,
        scratch_types=[
            pltpu.VMEM((ch, CHUNK), jnp.int32),
            pltpu.VMEM((ch, CHUNK), jnp.int32),
            pltpu.VMEM((CHUNK,), jnp.float32),
            pltpu.VMEM((NBUF, CHUNK, d), jnp.float32),
            pltpu.VMEM((NBUF, CHUNK, d), jnp.float32),
            pltpu.VMEM((CHUNK,), jnp.float32),
            pltpu.VMEM((LANES * LANES,), jnp.float32),
            pltpu.VMEM((2048,), jnp.float32),
            pltpu.VMEM_SHARED((n_pad,), jnp.float32),
            pltpu.VMEM_SHARED((n_pad,), jnp.float32),
            pltpu.SemaphoreType.DMA,
            pltpu.SemaphoreType.DMA,
        ],
    )
    def edge_kernel(xhat_hbm, src_hbm, dst_hbm, sums_hbm, degs_hbm,
                    src_v, dst_v, val_c, srows, drows, sim_v, tmp_v, stage_v,
                    shared_sum, shared_deg, sem_a, sem_b):
        cid = lax.axis_index("c")
        sid = lax.axis_index("s")
        wid = sid * NUM_CORES + cid
        zeros16 = jnp.zeros((LANES,), jnp.float32)
        lane_iota = lax.iota(jnp.int32, LANES)

        @pl.when(sid == 0)
        def _init():
            @pl.loop(0, 2048, step=LANES)
            def _z(i):
                stage_v[pl.ds(pl.multiple_of(i, LANES), LANES)] = zeros16

            @pl.loop(0, n_pad, step=2048)
            def _zs(i):
                ii = pl.multiple_of(i, 2048)
                pltpu.sync_copy(stage_v, shared_sum.at[pl.ds(ii, 2048)])
                pltpu.sync_copy(stage_v, shared_deg.at[pl.ds(ii, 2048)])

        pltpu.sync_copy(src_hbm.at[wid], src_v)
        pltpu.sync_copy(dst_hbm.at[wid], dst_v)

        for b in range(NBUF):
            pltpu.async_copy(xhat_hbm.at[src_v.at[b]], srows.at[b], sem_a)
            pltpu.async_copy(xhat_hbm.at[dst_v.at[b]], drows.at[b], sem_b)
        plsc.subcore_barrier()

        @pl.loop(0, ch, step=NBUF)
        def _ring(g):
            for b in range(NBUF):
                j = g + b
                srow = srows.at[b]
                drow = drows.at[b]
                pltpu.make_async_copy(xhat_hbm.at[src_v.at[j]], srow, sem_a).wait()
                pltpu.make_async_copy(xhat_hbm.at[dst_v.at[j]], drow, sem_b).wait()

                chunk_eid = (wid * ch + j) * CHUNK

                @pl.loop(0, CHUNK // LANES)
                def _group(g2):
                    base = pl.multiple_of(g2 * LANES, LANES)
                    vmask = jnp.where(chunk_eid + base + lane_iota < n_edges,
                                      1.0, 0.0).astype(jnp.float32)
                    for rr in range(LANES):
                        a = srow[base + rr, pl.ds(0, LANES)]
                        bb = drow[base + rr, pl.ds(0, LANES)]
                        acc = a * bb
                        for kk in range(1, nseg):
                            a = srow[base + rr, pl.ds(kk * LANES, LANES)]
                            bb = drow[base + rr, pl.ds(kk * LANES, LANES)]
                            acc = acc + a * bb
                        tmp_v[pl.ds(rr * LANES, LANES)] = acc
                    col = zeros16
                    for c in range(LANES):
                        rot = jnp.bitwise_and(lane_iota + c, LANES - 1)
                        col = col + plsc.load_gather(
                            tmp_v, [lane_iota * LANES + rot])
                    sim_v[pl.ds(base, LANES)] = col * vmask
                    val_c[pl.ds(base, LANES)] = vmask

                pltpu.sync_copy(sim_v, shared_sum.at[src_v.at[j]], add=True)
                pltpu.sync_copy(sim_v, shared_sum.at[dst_v.at[j]], add=True)
                pltpu.sync_copy(val_c, shared_deg.at[src_v.at[j]], add=True)
                pltpu.sync_copy(val_c, shared_deg.at[dst_v.at[j]], add=True)

                @pl.when(j + NBUF < ch)
                def _prefetch():
                    pltpu.async_copy(xhat_hbm.at[src_v.at[j + NBUF]], srows.at[b], sem_a)
                    pltpu.async_copy(xhat_hbm.at[dst_v.at[j + NBUF]], drows.at[b], sem_b)

        plsc.subcore_barrier()

        @pl.when(sid == 0)
        def _drain():
            @pl.loop(0, n_pad, step=2048)
            def _d(i):
                ii = pl.multiple_of(i, 2048)
                pltpu.sync_copy(shared_sum.at[pl.ds(ii, 2048)], stage_v)
                pltpu.sync_copy(stage_v, sums_hbm.at[cid].at[pl.ds(ii, 2048)])
                pltpu.sync_copy(shared_deg.at[pl.ds(ii, 2048)], stage_v)
                pltpu.sync_copy(stage_v, degs_hbm.at[cid].at[pl.ds(ii, 2048)])

    return edge_kernel


def kernel(node_features, edge_index):
    n, d = node_features.shape
    e = edge_index.shape[1]

    xhat = pl.pallas_call(
        _normalize_body,
        out_shape=jax.ShapeDtypeStruct((n, d), jnp.float32),
    )(node_features)

    ch = -(-e // (NW * CHUNK))
    ch = NBUF * (-(-ch // NBUF))
    ep = NW * CHUNK * ch
    pad = ep - e
    src = edge_index[0].astype(jnp.int32)
    dst = edge_index[1].astype(jnp.int32)
    srcp = jnp.pad(src, (0, pad)).reshape(NW, ch, CHUNK)
    dstp = jnp.pad(dst, (0, pad)).reshape(NW, ch, CHUNK)

    sums, degs = _make_edge_kernel(n, d, ch, e)(xhat, srcp, dstp)

    n_pad = sums.shape[1]
    out = pl.pallas_call(
        _finalize_body,
        out_shape=jax.ShapeDtypeStruct((1, n_pad), jnp.float32),
    )(sums, degs)
    return out.reshape(n_pad)[:n]
